# Initial kernel scaffold; baseline (speedup 1.0000x reference)
import functools

import jax
import jax.numpy as jnp
from jax import lax
from jax.experimental import pallas as pl
from jax.experimental.pallas import tpu as pltpu
from jax.experimental.pallas import tpu_sc as plsc

N_USERS = 25000
N_ITEMS = 75000
N_NODES = N_USERS + N_ITEMS
EMB = 32
NNZ = 1600000
B = 16384

NC = 2
NS = 16
L = 16

HALF = N_NODES // NC
TPT = HALF // NS
TRASH = HALF
ACC_ROWS = HALF + 8
CHUNK = 1024
CPT = 98
NNZ_PAD = NS * CPT * CHUNK
IDX_ROWS = NNZ_PAD // 128
RPT = B // (NC * NS)

_mesh = plsc.VectorSubcoreMesh(core_axis_name="c", subcore_axis_name="s")


@functools.partial(
    pl.kernel,
    out_type=jax.ShapeDtypeStruct((N_NODES, EMB), jnp.float32),
    mesh=_mesh,
    scratch_types=[
        pltpu.VMEM_SHARED((ACC_ROWS, EMB), jnp.float32),
        pltpu.VMEM((8, 128), jnp.int32),
        pltpu.VMEM((CHUNK,), jnp.int32),
        pltpu.VMEM((CHUNK,), jnp.float32),
        pltpu.VMEM((8, 128), jnp.int32),
        pltpu.VMEM((CHUNK, EMB), jnp.float32),
        pltpu.SemaphoreType.DMA,
    ],
)
def _spmm(ego, col2d, row1d, val1d, ax, acc, col_v, row_v, val_v, lidx_v,
          rows_v, sem):
    cid = lax.axis_index("c")
    sid = lax.axis_index("s")
    base = cid * HALF

    zeros = jnp.zeros((L,), jnp.float32)

    def _zero_body(r, carry):
        rows_v[r, pl.ds(0, L)] = zeros
        rows_v[r, pl.ds(L, L)] = zeros
        return carry

    lax.fori_loop(0, CHUNK, _zero_body, 0)
    for k in range(3):
        pltpu.sync_copy(rows_v.at[pl.ds(0, 1024)],
                        acc.at[pl.ds(sid * TPT + k * 1024, 1024)])
    pltpu.sync_copy(rows_v.at[pl.ds(0, TPT - 3072)],
                    acc.at[pl.ds(sid * TPT + 3072, TPT - 3072)])

    @pl.when(sid == 0)
    def _():
        pltpu.sync_copy(rows_v.at[pl.ds(0, ACC_ROWS - HALF)],
                        acc.at[pl.ds(HALF, ACC_ROWS - HALF)])

    plsc.subcore_barrier()

    def _chunk_body(ci, carry):
        cg = sid * CPT + ci
        eb = cg * CHUNK
        rb = cg * 8
        pltpu.sync_copy(col2d.at[pl.ds(rb, 8)], col_v)
        pltpu.sync_copy(row1d.at[pl.ds(eb, CHUNK)], row_v)
        pltpu.sync_copy(val1d.at[pl.ds(eb, CHUNK)], val_v)
        gathers = [
            pltpu.async_copy(ego.at[col_v.at[j]],
                             rows_v.at[pl.ds(j * 128, 128)], sem)
            for j in range(8)
        ]
        for g in range(64):
            r = row_v[pl.ds(g * L, L)]
            lcl = r - base
            ok = (r >= base) & (lcl < HALF)
            lidx_v[g // 8, pl.ds((g % 8) * L, L)] = jnp.where(ok, lcl, TRASH)
        for h in gathers:
            h.wait()

        def _scale_body(e4, c2):
            for u in range(4):
                e = e4 * 4 + u
                v = val_v[e]
                rows_v[e, pl.ds(0, L)] = rows_v[e, pl.ds(0, L)] * v
                rows_v[e, pl.ds(L, L)] = rows_v[e, pl.ds(L, L)] * v
            return c2

        lax.fori_loop(0, CHUNK // 4, _scale_body, 0)
        for j in range(8):
            pltpu.sync_copy(rows_v.at[pl.ds(j * 128, 128)],
                            acc.at[lidx_v.at[j]], add=True)
        return carry

    lax.fori_loop(0, CPT, _chunk_body, 0)

    plsc.subcore_barrier()
    dst0 = cid * HALF + sid * TPT
    for k in range(3):
        pltpu.sync_copy(acc.at[pl.ds(sid * TPT + k * 1024, 1024)],
                        ax.at[pl.ds(dst0 + k * 1024, 1024)])
    pltpu.sync_copy(acc.at[pl.ds(sid * TPT + 3072, TPT - 3072)],
                    ax.at[pl.ds(dst0 + 3072, TPT - 3072)])


@functools.partial(
    pl.kernel,
    out_type=(jax.ShapeDtypeStruct((B, EMB), jnp.float32),
              jax.ShapeDtypeStruct((B, EMB), jnp.float32)),
    mesh=_mesh,
    scratch_types=[
        pltpu.VMEM((4, 128), jnp.int32),
        pltpu.VMEM((4, 128), jnp.int32),
        pltpu.VMEM((RPT, EMB), jnp.float32),
        pltpu.VMEM((RPT, EMB), jnp.float32),
        pltpu.SemaphoreType.DMA,
    ],
)
def _combine(ax, utab, itab, uid2d, iid2d, u_out, i_out, id_v, idp_v,
             rows_e, rows_a, sem):
    cid = lax.axis_index("c")
    sid = lax.axis_index("s")
    wid = sid * NC + cid
    two = jnp.float32(2.0)

    def _side(tab, ids2d, out, off):
        pltpu.sync_copy(ids2d.at[pl.ds(wid * 4, 4)], id_v)
        for g in range(32):
            j, k = g // 8, (g % 8) * L
            idp_v[j, pl.ds(k, L)] = id_v[j, pl.ds(k, L)] + off
        hs = [
            pltpu.async_copy(tab.at[id_v.at[j]],
                             rows_e.at[pl.ds(j * 128, 128)], sem)
            for j in range(4)
        ]
        hs += [
            pltpu.async_copy(ax.at[idp_v.at[j]],
                             rows_a.at[pl.ds(j * 128, 128)], sem)
            for j in range(4)
        ]
        for h in hs:
            h.wait()

        def _comb_body(r4, c2):
            for u in range(4):
                r = r4 * 4 + u
                for hoff in (0, L):
                    rows_a[r, pl.ds(hoff, L)] = (
                        rows_a[r, pl.ds(hoff, L)]
                        + two * rows_e[r, pl.ds(hoff, L)])
            return c2

        lax.fori_loop(0, RPT // 4, _comb_body, 0)
        pltpu.sync_copy(rows_a, out.at[pl.ds(wid * RPT, RPT)])

    _side(utab, uid2d, u_out, 0)
    _side(itab, iid2d, i_out, N_USERS)


def kernel(user_table, item_table, adj_vals, adj_row, adj_col, user_id,
           item_id):
    ego = jnp.concatenate([user_table, item_table], axis=0)
    pad = NNZ_PAD - NNZ
    col_p = jnp.pad(adj_col.astype(jnp.int32), (0, pad)).reshape(IDX_ROWS, 128)
    row_p = jnp.pad(adj_row.astype(jnp.int32), (0, pad))
    val_p = jnp.pad(adj_vals, (0, pad))
    uid2d = user_id.astype(jnp.int32).reshape(B // 128, 128)
    iid2d = item_id.astype(jnp.int32).reshape(B // 128, 128)
    ax = _spmm(ego, col_p, row_p, val_p)
    u_embed, i_embed = _combine(ax, user_table, item_table, uid2d, iid2d)
    return (u_embed, i_embed)


# speedup vs baseline: 8.3418x; 8.3418x over previous
import functools

import jax
import jax.numpy as jnp
from jax import lax
from jax.experimental import pallas as pl
from jax.experimental.pallas import tpu as pltpu
from jax.experimental.pallas import tpu_sc as plsc

N_USERS = 25000
N_ITEMS = 75000
N_NODES = N_USERS + N_ITEMS
EMB = 32
NNZ = 1600000
B = 16384

NC = 2
NS = 16
L = 16

HALF = N_NODES // NC
ZPT = 3128
TRASH = HALF
ACC_ROWS = HALF + 8
CHUNK = 512
CPT = 196
NNZ_PAD = NS * CPT * CHUNK
IDX_ROWS = NNZ_PAD // 128
RPT = B // NS

_mesh = plsc.VectorSubcoreMesh(core_axis_name="c", subcore_axis_name="s")
_params = pltpu.CompilerParams(use_tc_tiling_on_sc=False)


@functools.partial(
    pl.kernel,
    out_type=jax.ShapeDtypeStruct((N_NODES, EMB), jnp.float32),
    mesh=_mesh,
    scratch_types=[
        pltpu.VMEM_SHARED((ACC_ROWS, EMB), jnp.float32),
        pltpu.VMEM((CHUNK // 128, 128), jnp.int32),
        pltpu.VMEM((CHUNK,), jnp.int32),
        pltpu.VMEM((CHUNK,), jnp.float32),
        pltpu.VMEM((CHUNK // 128, 128), jnp.int32),
        pltpu.VMEM((CHUNK, EMB), jnp.float32),
        pltpu.SemaphoreType.DMA,
    ],
    compiler_params=_params,
)
def _spmm(ego, col2d, row1d, val1d, ax, acc, col_v, row_v, val_v, lidx_v,
          rows_v, sem):
    cid = lax.axis_index("c")
    sid = lax.axis_index("s")
    base = cid * HALF

    zeros = jnp.zeros((L,), jnp.float32)

    def _zero_body(r, carry):
        rows_v[r, pl.ds(0, L)] = zeros
        rows_v[r, pl.ds(L, L)] = zeros
        return carry

    lax.fori_loop(0, CHUNK, _zero_body, 0)
    for k in range(6):
        pltpu.sync_copy(rows_v.at[pl.ds(0, CHUNK)],
                        acc.at[pl.ds(sid * ZPT + k * CHUNK, CHUNK)])

    @pl.when(sid < NS - 1)
    def _():
        pltpu.sync_copy(rows_v.at[pl.ds(0, ZPT - 3072)],
                        acc.at[pl.ds(sid * ZPT + 3072, ZPT - 3072)])

    @pl.when(sid == NS - 1)
    def _():
        pltpu.sync_copy(rows_v.at[pl.ds(0, 16)],
                        acc.at[pl.ds((NS - 1) * ZPT + 3072, 16)])

    plsc.subcore_barrier()

    def _chunk_body(ci, carry):
        cg = sid * CPT + ci
        eb = cg * CHUNK
        rb = cg * (CHUNK // 128)
        pltpu.sync_copy(col2d.at[pl.ds(rb, CHUNK // 128)], col_v)
        pltpu.sync_copy(row1d.at[pl.ds(eb, CHUNK)], row_v)
        pltpu.sync_copy(val1d.at[pl.ds(eb, CHUNK)], val_v)
        gathers = [
            pltpu.async_copy(ego.at[col_v.at[j]],
                             rows_v.at[pl.ds(j * 128, 128)], sem)
            for j in range(CHUNK // 128)
        ]
        for g in range(CHUNK // L):
            r = row_v[pl.ds(g * L, L)]
            lcl = r - base
            ok = (r >= base) & (lcl < HALF)
            lidx_v[g // 8, pl.ds((g % 8) * L, L)] = jnp.where(ok, lcl, TRASH)
        for h in gathers:
            h.wait()

        def _scale_body(g, c2):
            vv = val_v[pl.ds(g * L, L)]
            for u in range(L):
                e = g * L + u
                s = vv[u]
                rows_v[e, pl.ds(0, L)] = rows_v[e, pl.ds(0, L)] * s
                rows_v[e, pl.ds(L, L)] = rows_v[e, pl.ds(L, L)] * s
            return c2

        lax.fori_loop(0, CHUNK // L, _scale_body, 0)
        for j in range(CHUNK // 128):
            pltpu.sync_copy(rows_v.at[pl.ds(j * 128, 128)],
                            acc.at[lidx_v.at[j]], add=True)
        return carry

    lax.fori_loop(0, CPT, _chunk_body, 0)

    plsc.subcore_barrier()
    dst0 = cid * HALF + sid * ZPT
    for k in range(3):
        pltpu.sync_copy(acc.at[pl.ds(sid * ZPT + k * 1024, 1024)],
                        ax.at[pl.ds(dst0 + k * 1024, 1024)])

    @pl.when(sid < NS - 1)
    def _():
        pltpu.sync_copy(acc.at[pl.ds(sid * ZPT + 3072, ZPT - 3072)],
                        ax.at[pl.ds(dst0 + 3072, ZPT - 3072)])

    @pl.when(sid == NS - 1)
    def _():
        pltpu.sync_copy(acc.at[pl.ds((NS - 1) * ZPT + 3072, 8)],
                        ax.at[pl.ds(cid * HALF + (NS - 1) * ZPT + 3072, 8)])


@functools.partial(
    pl.kernel,
    out_type=(jax.ShapeDtypeStruct((B, EMB), jnp.float32),
              jax.ShapeDtypeStruct((B, EMB), jnp.float32)),
    mesh=_mesh,
    scratch_types=[
        pltpu.VMEM((8, 128), jnp.int32),
        pltpu.VMEM((8, 128), jnp.int32),
        pltpu.VMEM((RPT, EMB), jnp.float32),
        pltpu.VMEM((RPT, EMB), jnp.float32),
        pltpu.SemaphoreType.DMA,
    ],
    compiler_params=_params,
)
def _combine(ax, utab, itab, uid2d, iid2d, u_out, i_out, id_v, idp_v,
             rows_e, rows_a, sem):
    cid = lax.axis_index("c")
    sid = lax.axis_index("s")
    two = jnp.float32(2.0)

    def _side(tab, ids2d, out, off):
        pltpu.sync_copy(ids2d.at[pl.ds(sid * 8, 8)], id_v)
        for g in range(64):
            j, k = g // 8, (g % 8) * L
            idp_v[j, pl.ds(k, L)] = id_v[j, pl.ds(k, L)] + off
        hs = [
            pltpu.async_copy(tab.at[id_v.at[j]],
                             rows_e.at[pl.ds(j * 128, 128)], sem)
            for j in range(8)
        ]
        hs += [
            pltpu.async_copy(ax.at[idp_v.at[j]],
                             rows_a.at[pl.ds(j * 128, 128)], sem)
            for j in range(8)
        ]
        for h in hs:
            h.wait()

        def _comb_body(r4, c2):
            for u in range(4):
                r = r4 * 4 + u
                for hoff in (0, L):
                    rows_a[r, pl.ds(hoff, L)] = (
                        rows_a[r, pl.ds(hoff, L)]
                        + two * rows_e[r, pl.ds(hoff, L)])
            return c2

        lax.fori_loop(0, RPT // 4, _comb_body, 0)
        pltpu.sync_copy(rows_a, out.at[pl.ds(sid * RPT, RPT)])

    @pl.when(cid == 0)
    def _():
        _side(utab, uid2d, u_out, 0)

    @pl.when(cid == 1)
    def _():
        _side(itab, iid2d, i_out, N_USERS)


def kernel(user_table, item_table, adj_vals, adj_row, adj_col, user_id,
           item_id):
    ego = jnp.concatenate([user_table, item_table], axis=0)
    pad = NNZ_PAD - NNZ
    col_p = jnp.pad(adj_col.astype(jnp.int32), (0, pad)).reshape(IDX_ROWS, 128)
    row_p = jnp.pad(adj_row.astype(jnp.int32), (0, pad))
    val_p = jnp.pad(adj_vals, (0, pad))
    uid2d = user_id.astype(jnp.int32).reshape(B // 128, 128)
    iid2d = item_id.astype(jnp.int32).reshape(B // 128, 128)
    ax = _spmm(ego, col_p, row_p, val_p)
    u_embed, i_embed = _combine(ax, user_table, item_table, uid2d, iid2d)
    return (u_embed, i_embed)


# speedup vs baseline: 12.4535x; 1.4929x over previous
import functools

import jax
import jax.numpy as jnp
from jax import lax
from jax.experimental import pallas as pl
from jax.experimental.pallas import tpu as pltpu
from jax.experimental.pallas import tpu_sc as plsc

N_USERS = 25000
N_ITEMS = 75000
N_NODES = N_USERS + N_ITEMS
EMB = 32
NNZ = 1600000
B = 16384

NC = 2
NS = 16
L = 16

HALF = N_NODES // NC
ZPT = 3128
TRASH = HALF
ACC_ROWS = HALF + 8
CHUNK = 1024
CPT = 98
GPC = CHUNK // 128
FCH = 512
NNZ_PAD = NS * CPT * CHUNK
IDX_ROWS = NNZ_PAD // 128
RPT = B // NS

_mesh = plsc.VectorSubcoreMesh(core_axis_name="c", subcore_axis_name="s")
_params = pltpu.CompilerParams(use_tc_tiling_on_sc=False,
                               needs_layout_passes=False)


@functools.partial(
    pl.kernel,
    out_type=jax.ShapeDtypeStruct((N_NODES, EMB), jnp.float32),
    mesh=_mesh,
    scratch_types=[
        pltpu.VMEM_SHARED((ACC_ROWS, EMB), jnp.bfloat16),
        pltpu.VMEM((GPC, 128), jnp.int32),
        pltpu.VMEM((GPC, 128), jnp.int32),
        pltpu.VMEM((CHUNK,), jnp.int32),
        pltpu.VMEM((CHUNK,), jnp.int32),
        pltpu.VMEM((CHUNK,), jnp.float32),
        pltpu.VMEM((CHUNK,), jnp.float32),
        pltpu.VMEM((GPC, 128), jnp.int32),
        pltpu.VMEM((GPC, 128), jnp.int32),
        pltpu.VMEM((CHUNK, EMB), jnp.bfloat16),
        pltpu.VMEM((CHUNK, EMB), jnp.bfloat16),
        pltpu.VMEM((FCH, EMB), jnp.float32),
        pltpu.SemaphoreType.DMA,
        pltpu.SemaphoreType.DMA,
        pltpu.SemaphoreType.DMA,
        pltpu.SemaphoreType.DMA,
        pltpu.SemaphoreType.DMA,
        pltpu.SemaphoreType.DMA,
    ],
    compiler_params=_params,
)
def _spmm(ego, col2d, row1d, val1d, ax,
          acc, col0, col1, rowv0, rowv1, valv0, valv1, lidx0, lidx1,
          rbf0, rbf1, fout, sl0, sl1, sg0, sg1, ss0, ss1):
    cid = lax.axis_index("c")
    sid = lax.axis_index("s")
    base = cid * HALF
    cols = (col0, col1)
    rows = (rowv0, rowv1)
    vals = (valv0, valv1)
    lidxs = (lidx0, lidx1)
    rbfs = (rbf0, rbf1)
    sls = (sl0, sl1)
    sgs = (sg0, sg1)
    sss = (ss0, ss1)

    zb = jnp.zeros((2 * L,), jnp.bfloat16)

    def _zero_body(r, carry):
        rbf0[r, pl.ds(0, 2 * L)] = zb
        return carry

    lax.fori_loop(0, CHUNK, _zero_body, 0)
    for k in range(3):
        pltpu.sync_copy(rbf0.at[pl.ds(0, 1024)],
                        acc.at[pl.ds(sid * ZPT + k * 1024, 1024)])

    @pl.when(sid < NS - 1)
    def _():
        pltpu.sync_copy(rbf0.at[pl.ds(0, ZPT - 3072)],
                        acc.at[pl.ds(sid * ZPT + 3072, ZPT - 3072)])

    @pl.when(sid == NS - 1)
    def _():
        pltpu.sync_copy(rbf0.at[pl.ds(0, 16)],
                        acc.at[pl.ds((NS - 1) * ZPT + 3072, 16)])

    plsc.subcore_barrier()

    def _issue_loads(ci, p):
        cg = sid * CPT + ci
        pltpu.async_copy(col2d.at[pl.ds(cg * GPC, GPC)], cols[p], sls[p])
        pltpu.async_copy(row1d.at[pl.ds(cg * CHUNK, CHUNK)], rows[p], sls[p])
        pltpu.async_copy(val1d.at[pl.ds(cg * CHUNK, CHUNK)], vals[p], sls[p])

    def _wait_loads(p):
        pltpu.make_async_copy(col2d.at[pl.ds(0, GPC)], cols[p], sls[p]).wait()
        pltpu.make_async_copy(row1d.at[pl.ds(0, CHUNK)], rows[p],
                              sls[p]).wait()
        pltpu.make_async_copy(val1d.at[pl.ds(0, CHUNK)], vals[p],
                              sls[p]).wait()

    def _issue_gathers(p):
        for j in range(GPC):
            pltpu.async_copy(ego.at[cols[p].at[j]],
                             rbfs[p].at[pl.ds(j * 128, 128)], sgs[p])

    def _wait_gathers(p):
        for j in range(GPC):
            pltpu.make_async_copy(ego.at[cols[p].at[j]],
                                  rbfs[p].at[pl.ds(j * 128, 128)],
                                  sgs[p]).wait()

    def _issue_scatters(p):
        for j in range(GPC):
            pltpu.async_copy(rbfs[p].at[pl.ds(j * 128, 128)],
                             acc.at[lidxs[p].at[j]], sss[p], add=True)

    def _drain_scatters(p):
        for j in range(GPC):
            pltpu.make_async_copy(rbfs[p].at[pl.ds(j * 128, 128)],
                                  acc.at[lidxs[p].at[j]], sss[p]).wait()

    _issue_loads(0, 0)
    _issue_loads(1, 1)
    _wait_loads(0)
    _issue_gathers(0)

    def _iter(ci, p):
        q = 1 - p

        @pl.when(ci + 1 < CPT)
        def _():
            _wait_loads(q)

            @pl.when(ci >= 1)
            def _():
                _drain_scatters(q)

            _issue_gathers(q)

        for g in range(CHUNK // L):
            r = rows[p][pl.ds(g * L, L)]
            lcl = r - base
            ok = (r >= base) & (lcl < HALF)
            lidxs[p][g // 8, pl.ds((g % 8) * L, L)] = jnp.where(ok, lcl,
                                                               TRASH)

        _wait_gathers(p)

        def _scale_body(g, c2):
            vv = vals[p][pl.ds(g * L, L)]
            for u in range(L):
                e = g * L + u
                sv = lax.broadcast_in_dim(vv[u], (L,), ())
                sb = plsc.pack(sv, sv, format=plsc.PackFormat.INTERLEAVED)
                rbfs[p][e, pl.ds(0, 2 * L)] = (
                    rbfs[p][e, pl.ds(0, 2 * L)] * sb)
            return c2

        lax.fori_loop(0, CHUNK // L, _scale_body, 0)
        _issue_scatters(p)

        @pl.when(ci + 2 < CPT)
        def _():
            _issue_loads(ci + 2, p)

    def _outer(c2i, carry):
        _iter(c2i * 2, 0)
        _iter(c2i * 2 + 1, 1)
        return carry

    lax.fori_loop(0, CPT // 2, _outer, 0)
    _drain_scatters(0)
    _drain_scatters(1)
    plsc.subcore_barrier()

    ecols = jnp.arange(L, dtype=jnp.int32) * 2
    ocols = ecols + 1
    himask = jnp.full((L,), -65536, jnp.int32)

    def _conv_chunk(local0, hbm0, nrows):
        pltpu.sync_copy(acc.at[pl.ds(local0, nrows)],
                        rbf0.at[pl.ds(0, nrows)])

        def _cb(r, c2):
            w = plsc.bitcast(rbf0[r, pl.ds(0, 2 * L)], jnp.int32)
            ev = plsc.bitcast(w << 16, jnp.float32)
            od = plsc.bitcast(w & himask, jnp.float32)
            rr = jnp.full((L,), r, jnp.int32)
            plsc.store_scatter(fout, [rr, ecols], ev)
            plsc.store_scatter(fout, [rr, ocols], od)
            return c2

        lax.fori_loop(0, nrows, _cb, 0)
        pltpu.sync_copy(fout.at[pl.ds(0, nrows)], ax.at[pl.ds(hbm0, nrows)])

    loc0 = sid * ZPT
    dst0 = cid * HALF + sid * ZPT
    for k in range(6):
        _conv_chunk(loc0 + k * FCH, dst0 + k * FCH, FCH)

    @pl.when(sid < NS - 1)
    def _():
        _conv_chunk(loc0 + 3072, dst0 + 3072, ZPT - 3072)

    @pl.when(sid == NS - 1)
    def _():
        _conv_chunk(loc0 + 3072, dst0 + 3072, 8)


@functools.partial(
    pl.kernel,
    out_type=(jax.ShapeDtypeStruct((B, EMB), jnp.float32),
              jax.ShapeDtypeStruct((B, EMB), jnp.float32)),
    mesh=_mesh,
    scratch_types=[
        pltpu.VMEM((8, 128), jnp.int32),
        pltpu.VMEM((8, 128), jnp.int32),
        pltpu.VMEM((RPT, EMB), jnp.float32),
        pltpu.VMEM((RPT, EMB), jnp.float32),
        pltpu.SemaphoreType.DMA,
    ],
    compiler_params=_params,
)
def _combine(ax, utab, itab, uid2d, iid2d, u_out, i_out, id_v, idp_v,
             rows_e, rows_a, sem):
    cid = lax.axis_index("c")
    sid = lax.axis_index("s")
    two = jnp.float32(2.0)

    def _side(tab, ids2d, out, off):
        pltpu.sync_copy(ids2d.at[pl.ds(sid * 8, 8)], id_v)
        for g in range(64):
            j, k = g // 8, (g % 8) * L
            idp_v[j, pl.ds(k, L)] = id_v[j, pl.ds(k, L)] + off
        hs = [
            pltpu.async_copy(tab.at[id_v.at[j]],
                             rows_e.at[pl.ds(j * 128, 128)], sem)
            for j in range(8)
        ]
        hs += [
            pltpu.async_copy(ax.at[idp_v.at[j]],
                             rows_a.at[pl.ds(j * 128, 128)], sem)
            for j in range(8)
        ]
        for h in hs:
            h.wait()

        def _comb_body(r4, c2):
            for u in range(4):
                r = r4 * 4 + u
                for hoff in (0, L):
                    rows_a[r, pl.ds(hoff, L)] = (
                        rows_a[r, pl.ds(hoff, L)]
                        + two * rows_e[r, pl.ds(hoff, L)])
            return c2

        lax.fori_loop(0, RPT // 4, _comb_body, 0)
        pltpu.sync_copy(rows_a, out.at[pl.ds(sid * RPT, RPT)])

    @pl.when(cid == 0)
    def _():
        _side(utab, uid2d, u_out, 0)

    @pl.when(cid == 1)
    def _():
        _side(itab, iid2d, i_out, N_USERS)


def kernel(user_table, item_table, adj_vals, adj_row, adj_col, user_id,
           item_id):
    ego = jnp.concatenate([user_table, item_table],
                          axis=0).astype(jnp.bfloat16)
    pad = NNZ_PAD - NNZ
    col_p = jnp.pad(adj_col.astype(jnp.int32), (0, pad)).reshape(IDX_ROWS, 128)
    row_p = jnp.pad(adj_row.astype(jnp.int32), (0, pad))
    val_p = jnp.pad(adj_vals, (0, pad))
    uid2d = user_id.astype(jnp.int32).reshape(B // 128, 128)
    iid2d = item_id.astype(jnp.int32).reshape(B // 128, 128)
    ax = _spmm(ego, col_p, row_p, val_p)
    u_embed, i_embed = _combine(ax, user_table, item_table, uid2d, iid2d)
    return (u_embed, i_embed)


# speedup vs baseline: 23.8091x; 1.9118x over previous
import functools

import jax
import jax.numpy as jnp
from jax import lax
from jax.experimental import pallas as pl
from jax.experimental.pallas import tpu as pltpu
from jax.experimental.pallas import tpu_sc as plsc

N_USERS = 25000
N_ITEMS = 75000
N_NODES = N_USERS + N_ITEMS
EMB = 32
NNZ = 1600000
B = 16384

NC = 2
NS = 16
L = 16

HALF = N_NODES // NC
ZPT = 3128
TRASH = HALF
ACC_ROWS = HALF + 16
CHUNK = 1024
CPT = 98
GPC = CHUNK // 128
FCH = 512
NNZ_PAD = NS * CPT * CHUNK
IDX_ROWS = NNZ_PAD // 128
RPT = B // NS

_mesh = plsc.VectorSubcoreMesh(core_axis_name="c", subcore_axis_name="s")
_params = pltpu.CompilerParams(use_tc_tiling_on_sc=False,
                               needs_layout_passes=False)


@functools.partial(
    pl.kernel,
    out_type=jax.ShapeDtypeStruct((N_NODES, EMB), jnp.float32),
    mesh=_mesh,
    scratch_types=[
        pltpu.VMEM_SHARED((ACC_ROWS, EMB), jnp.bfloat16),
        pltpu.VMEM((GPC, 128), jnp.int32),
        pltpu.VMEM((GPC, 128), jnp.int32),
        pltpu.VMEM((CHUNK,), jnp.int32),
        pltpu.VMEM((CHUNK,), jnp.int32),
        pltpu.VMEM((CHUNK,), jnp.float32),
        pltpu.VMEM((CHUNK,), jnp.float32),
        pltpu.VMEM((GPC, 128), jnp.int32),
        pltpu.VMEM((GPC, 128), jnp.int32),
        pltpu.VMEM((CHUNK, EMB), jnp.bfloat16),
        pltpu.VMEM((CHUNK, EMB), jnp.bfloat16),
        pltpu.VMEM((FCH, EMB), jnp.float32),
        pltpu.SemaphoreType.DMA,
        pltpu.SemaphoreType.DMA,
        pltpu.SemaphoreType.DMA,
        pltpu.SemaphoreType.DMA,
        pltpu.SemaphoreType.DMA,
        pltpu.SemaphoreType.DMA,
    ],
    compiler_params=_params,
)
def _spmm(ego, col2d, row1d, val1d, ax,
          acc, col0, col1, rowv0, rowv1, valv0, valv1, lidx0, lidx1,
          rbf0, rbf1, fout, sl0, sl1, sg0, sg1, ss0, ss1):
    cid = lax.axis_index("c")
    sid = lax.axis_index("s")
    base = cid * HALF
    cols = (col0, col1)
    rows = (rowv0, rowv1)
    vals = (valv0, valv1)
    lidxs = (lidx0, lidx1)
    rbfs = (rbf0, rbf1)
    sls = (sl0, sl1)
    sgs = (sg0, sg1)
    sss = (ss0, ss1)

    zb = jnp.zeros((2 * L,), jnp.bfloat16)

    def _zero_body(r, carry):
        rbf0[r, pl.ds(0, 2 * L)] = zb
        return carry

    lax.fori_loop(0, CHUNK, _zero_body, 0)
    for k in range(3):
        pltpu.sync_copy(rbf0.at[pl.ds(0, 1024)],
                        acc.at[pl.ds(sid * ZPT + k * 1024, 1024)])

    @pl.when(sid < NS - 1)
    def _():
        pltpu.sync_copy(rbf0.at[pl.ds(0, ZPT - 3072)],
                        acc.at[pl.ds(sid * ZPT + 3072, ZPT - 3072)])

    @pl.when(sid == NS - 1)
    def _():
        pltpu.sync_copy(rbf0.at[pl.ds(0, 24)],
                        acc.at[pl.ds((NS - 1) * ZPT + 3072, 24)])

    plsc.subcore_barrier()

    def _issue_loads(ci, p):
        cg = sid * CPT + ci
        pltpu.async_copy(col2d.at[pl.ds(cg * GPC, GPC)], cols[p], sls[p])
        pltpu.async_copy(row1d.at[pl.ds(cg * CHUNK, CHUNK)], rows[p], sls[p])
        pltpu.async_copy(val1d.at[pl.ds(cg * CHUNK, CHUNK)], vals[p], sls[p])

    def _wait_loads(p):
        pltpu.make_async_copy(col2d.at[pl.ds(0, GPC)], cols[p], sls[p]).wait()
        pltpu.make_async_copy(row1d.at[pl.ds(0, CHUNK)], rows[p],
                              sls[p]).wait()
        pltpu.make_async_copy(val1d.at[pl.ds(0, CHUNK)], vals[p],
                              sls[p]).wait()

    def _issue_gathers(p):
        for j in range(GPC):
            pltpu.async_copy(ego.at[cols[p].at[j]],
                             rbfs[p].at[pl.ds(j * 128, 128)], sgs[p])

    def _wait_gathers(p):
        for j in range(GPC):
            pltpu.make_async_copy(ego.at[cols[p].at[j]],
                                  rbfs[p].at[pl.ds(j * 128, 128)],
                                  sgs[p]).wait()

    def _issue_scatters(p):
        for j in range(GPC):
            pltpu.async_copy(rbfs[p].at[pl.ds(j * 128, 128)],
                             acc.at[lidxs[p].at[j]], sss[p], add=True)

    def _drain_scatters(p):
        for j in range(GPC):
            pltpu.make_async_copy(rbfs[p].at[pl.ds(j * 128, 128)],
                                  acc.at[lidxs[p].at[j]], sss[p]).wait()

    _issue_loads(0, 0)
    _issue_loads(1, 1)
    _wait_loads(0)
    _issue_gathers(0)

    def _iter(ci, p):
        q = 1 - p

        @pl.when(ci + 1 < CPT)
        def _():
            _wait_loads(q)

            @pl.when(ci >= 1)
            def _():
                _drain_scatters(q)

            _issue_gathers(q)

        for g in range(CHUNK // L):
            r = rows[p][pl.ds(g * L, L)]
            lcl = r - base
            ok = (r >= base) & (lcl < HALF)
            lidxs[p][g // 8, pl.ds((g % 8) * L, L)] = jnp.where(
                ok, lcl, TRASH + (g % 16))

        _wait_gathers(p)

        def _scale_body(g, c2):
            vv = vals[p][pl.ds(g * L, L)]
            for u in range(L):
                e = g * L + u
                sv = lax.broadcast_in_dim(vv[u], (L,), ())
                sb = plsc.pack(sv, sv, format=plsc.PackFormat.INTERLEAVED)
                rbfs[p][e, pl.ds(0, 2 * L)] = (
                    rbfs[p][e, pl.ds(0, 2 * L)] * sb)
            return c2

        lax.fori_loop(0, CHUNK // L, _scale_body, 0)
        _issue_scatters(p)

        @pl.when(ci + 2 < CPT)
        def _():
            _issue_loads(ci + 2, p)

    def _outer(c2i, carry):
        _iter(c2i * 2, 0)
        _iter(c2i * 2 + 1, 1)
        return carry

    lax.fori_loop(0, CPT // 2, _outer, 0)
    _drain_scatters(0)
    _drain_scatters(1)
    plsc.subcore_barrier()

    ecols = jnp.arange(L, dtype=jnp.int32) * 2
    ocols = ecols + 1
    himask = jnp.full((L,), -65536, jnp.int32)

    def _conv_chunk(local0, hbm0, nrows):
        pltpu.sync_copy(acc.at[pl.ds(local0, nrows)],
                        rbf0.at[pl.ds(0, nrows)])

        def _cb(r, c2):
            w = plsc.bitcast(rbf0[r, pl.ds(0, 2 * L)], jnp.int32)
            ev = plsc.bitcast(w << 16, jnp.float32)
            od = plsc.bitcast(w & himask, jnp.float32)
            rr = jnp.full((L,), r, jnp.int32)
            plsc.store_scatter(fout, [rr, ecols], ev)
            plsc.store_scatter(fout, [rr, ocols], od)
            return c2

        lax.fori_loop(0, nrows, _cb, 0)
        pltpu.sync_copy(fout.at[pl.ds(0, nrows)], ax.at[pl.ds(hbm0, nrows)])

    loc0 = sid * ZPT
    dst0 = cid * HALF + sid * ZPT
    for k in range(6):
        _conv_chunk(loc0 + k * FCH, dst0 + k * FCH, FCH)

    @pl.when(sid < NS - 1)
    def _():
        _conv_chunk(loc0 + 3072, dst0 + 3072, ZPT - 3072)

    @pl.when(sid == NS - 1)
    def _():
        _conv_chunk(loc0 + 3072, dst0 + 3072, 8)


@functools.partial(
    pl.kernel,
    out_type=(jax.ShapeDtypeStruct((B, EMB), jnp.float32),
              jax.ShapeDtypeStruct((B, EMB), jnp.float32)),
    mesh=_mesh,
    scratch_types=[
        pltpu.VMEM((8, 128), jnp.int32),
        pltpu.VMEM((8, 128), jnp.int32),
        pltpu.VMEM((RPT, EMB), jnp.float32),
        pltpu.VMEM((RPT, EMB), jnp.float32),
        pltpu.SemaphoreType.DMA,
    ],
    compiler_params=_params,
)
def _combine(ax, utab, itab, uid2d, iid2d, u_out, i_out, id_v, idp_v,
             rows_e, rows_a, sem):
    cid = lax.axis_index("c")
    sid = lax.axis_index("s")
    two = jnp.float32(2.0)

    def _side(tab, ids2d, out, off):
        pltpu.sync_copy(ids2d.at[pl.ds(sid * 8, 8)], id_v)
        for g in range(64):
            j, k = g // 8, (g % 8) * L
            idp_v[j, pl.ds(k, L)] = id_v[j, pl.ds(k, L)] + off
        hs = [
            pltpu.async_copy(tab.at[id_v.at[j]],
                             rows_e.at[pl.ds(j * 128, 128)], sem)
            for j in range(8)
        ]
        hs += [
            pltpu.async_copy(ax.at[idp_v.at[j]],
                             rows_a.at[pl.ds(j * 128, 128)], sem)
            for j in range(8)
        ]
        for h in hs:
            h.wait()

        def _comb_body(r4, c2):
            for u in range(4):
                r = r4 * 4 + u
                for hoff in (0, L):
                    rows_a[r, pl.ds(hoff, L)] = (
                        rows_a[r, pl.ds(hoff, L)]
                        + two * rows_e[r, pl.ds(hoff, L)])
            return c2

        lax.fori_loop(0, RPT // 4, _comb_body, 0)
        pltpu.sync_copy(rows_a, out.at[pl.ds(sid * RPT, RPT)])

    @pl.when(cid == 0)
    def _():
        _side(utab, uid2d, u_out, 0)

    @pl.when(cid == 1)
    def _():
        _side(itab, iid2d, i_out, N_USERS)


def kernel(user_table, item_table, adj_vals, adj_row, adj_col, user_id,
           item_id):
    ego = jnp.concatenate([user_table, item_table],
                          axis=0).astype(jnp.bfloat16)
    pad = NNZ_PAD - NNZ
    col_p = jnp.pad(adj_col.astype(jnp.int32), (0, pad)).reshape(IDX_ROWS, 128)
    row_p = jnp.pad(adj_row.astype(jnp.int32), (0, pad))
    val_p = jnp.pad(adj_vals, (0, pad))
    uid2d = user_id.astype(jnp.int32).reshape(B // 128, 128)
    iid2d = item_id.astype(jnp.int32).reshape(B // 128, 128)
    ax = _spmm(ego, col_p, row_p, val_p)
    u_embed, i_embed = _combine(ax, user_table, item_table, uid2d, iid2d)
    return (u_embed, i_embed)
